# baseline (speedup 1.0000x reference)
import functools

import jax
import jax.numpy as jnp
import numpy as np
from jax.experimental import pallas as pl

J = 17
H = 8
C = 16
F = H * C
KP = 16
WF = KP * F
KH = KP * H
_LOG2E = 1.4426950408889634

_EXP = np.kron(np.eye(H, dtype=np.float32), np.ones((1, C), dtype=np.float32))
_EXP_W = np.kron(np.eye(KP, dtype=np.float32), _EXP)


def _elu(z):
    return jnp.where(z > 0, z, jnp.exp2(jnp.minimum(z, 0.0) * _LOG2E) - 1.0)


def _widen(x3, R):
    return jnp.concatenate([x3[:, k * R:(k + 1) * R, :] for k in range(KP)],
                           axis=-1)


def _narrow(hw, R):
    return jnp.concatenate([hw[:, :, k * F:(k + 1) * F] for k in range(KP)],
                           axis=1)


def _gat_wide(x2, Wlr, blr, Aw, bias_w, EXPW, Gb):
    R = Gb // KP
    xlr = jnp.dot(x2, Wlr, preferred_element_type=jnp.float32) + blr
    xl = xlr[:, :F]
    xr = xlr[:, F:]
    xl_w = _widen(xl.reshape(J, Gb, F), R)
    xr_w = _widen(xr.reshape(J, Gb, F), R)
    rows = []
    for i in range(J):
        u = xl_w + xr_w[i][None]
        e = jnp.maximum(u, 0.2 * u)
        logits = jnp.dot(e.reshape(J * R, WF), Aw,
                         preferred_element_type=jnp.float32)
        l3 = logits.reshape(J, R, KH)
        m = jnp.max(l3, axis=0, keepdims=True)
        p = jnp.exp2(l3 - m)
        s = jnp.sum(p, axis=0, keepdims=True)
        alpha = p * (1.0 / (s + 1e-16))
        a128 = jnp.dot(alpha.reshape(J * R, KH), EXPW,
                       preferred_element_type=jnp.float32).reshape(J, R, WF)
        rows.append(jnp.sum(a128 * xl_w, axis=0))
    return jnp.stack(rows, axis=0) + bias_w


def _body(x_ref, Wlr1_ref, blr1_ref, A1_ref, bias1_ref,
          Wlr2_ref, blr2_ref, A2_ref, bias2_ref,
          EXPW_ref, WoutR_ref, bout_ref, out_ref, *, Gb):
    R = Gb // KP
    EXPW = EXPW_ref[...]
    x = x_ref[...]
    h1w = _gat_wide(x.reshape(J * Gb, 3),
                    Wlr1_ref[...], blr1_ref[...],
                    A1_ref[...], bias1_ref[...], EXPW, Gb)
    h1 = _narrow(_elu(h1w), R)
    h2w = _gat_wide(h1.reshape(J * Gb, F),
                    Wlr2_ref[...], blr2_ref[...],
                    A2_ref[...], bias2_ref[...], EXPW, Gb)
    h2 = _narrow(_elu(h2w), R)
    acc = jnp.dot(h2[0], WoutR_ref[0], preferred_element_type=jnp.float32)
    for j in range(1, J):
        acc = acc + jnp.dot(h2[j], WoutR_ref[j],
                            preferred_element_type=jnp.float32)
    out_ref[...] = acc + bout_ref[...]


def kernel(x_seq, Wl1, bl1, Wr1, br1, att1, bias1,
           Wl2, bl2, Wr2, br2, att2, bias2, Wout, bout):
    B, T, _ = x_seq.shape
    G = B * T
    Gb = 256
    assert G % Gb == 0

    x3 = x_seq.reshape(G, J, 3).transpose(1, 0, 2)

    EXP = jnp.asarray(_EXP)
    EXPW = jnp.asarray(_EXP_W)
    A1 = att1.reshape(F, 1) * EXP.T * _LOG2E
    A2 = att2.reshape(F, 1) * EXP.T * _LOG2E
    A1w = jnp.kron(jnp.eye(KP, dtype=jnp.float32), A1)
    A2w = jnp.kron(jnp.eye(KP, dtype=jnp.float32), A2)
    bias1w = jnp.tile(bias1, (KP,)).reshape(1, 1, WF)
    bias2w = jnp.tile(bias2, (KP,)).reshape(1, 1, WF)
    WoutR = Wout.reshape(J, F, 256)

    full = lambda shape: pl.BlockSpec(shape, lambda g: (0,) * len(shape))
    out = pl.pallas_call(
        functools.partial(_body, Gb=Gb),
        grid=(G // Gb,),
        in_specs=[
            pl.BlockSpec((J, Gb, 3), lambda g: (0, g, 0)),
            full((3, 2 * F)), full((1, 2 * F)),
            full((WF, KH)), full((1, 1, WF)),
            full((F, 2 * F)), full((1, 2 * F)),
            full((WF, KH)), full((1, 1, WF)),
            full((KH, WF)), full((J, F, 256)), full((1, 256)),
        ],
        out_specs=pl.BlockSpec((Gb, 256), lambda g: (g, 0)),
        out_shape=jax.ShapeDtypeStruct((G, 256), jnp.float32),
    )(x3, jnp.concatenate([Wl1, Wr1], axis=1),
      jnp.concatenate([bl1, br1]).reshape(1, 2 * F), A1w, bias1w,
      jnp.concatenate([Wl2, Wr2], axis=1),
      jnp.concatenate([bl2, br2]).reshape(1, 2 * F), A2w, bias2w,
      EXPW, WoutR, bout.reshape(1, 256))
    return out.reshape(B, T, 256)


# speedup vs baseline: 1.0139x; 1.0139x over previous
import functools

import jax
import jax.numpy as jnp
import numpy as np
from jax.experimental import pallas as pl

J = 17
H = 8
C = 16
F = H * C
KP = 16
WF = KP * F
KH = KP * H
_LOG2E = 1.4426950408889634

_EXP = np.kron(np.eye(H, dtype=np.float32), np.ones((1, C), dtype=np.float32))
_EXP_W = np.kron(np.eye(KP, dtype=np.float32), _EXP)


def _elu(z):
    return jnp.where(z > 0, z, jnp.exp2(jnp.minimum(z, 0.0) * _LOG2E) - 1.0)


def _widen(x3, R):
    return jnp.concatenate([x3[:, k * R:(k + 1) * R, :] for k in range(KP)],
                           axis=-1)


def _narrow(hw, R):
    return jnp.concatenate([hw[:, :, k * F:(k + 1) * F] for k in range(KP)],
                           axis=1)


def _gat_wide(x2, Wl, bl, Wr, br, Aw, bias_w, EXPW, Gb):
    R = Gb // KP
    xl = jnp.dot(x2, Wl, preferred_element_type=jnp.float32) + bl
    xr = jnp.dot(x2, Wr, preferred_element_type=jnp.float32) + br
    xl_w = _widen(xl.reshape(J, Gb, F), R)
    xr_w = _widen(xr.reshape(J, Gb, F), R)
    rows = []
    for i in range(J):
        u = xl_w + xr_w[i][None]
        e = jnp.maximum(u, 0.2 * u)
        logits = jnp.dot(e.reshape(J * R, WF), Aw,
                         preferred_element_type=jnp.float32)
        l3 = logits.reshape(J, R, KH)
        m = jnp.max(l3, axis=0, keepdims=True)
        p = jnp.exp2(l3 - m)
        s = jnp.sum(p, axis=0, keepdims=True)
        alpha = p * (1.0 / (s + 1e-16))
        a128 = jnp.dot(alpha.reshape(J * R, KH), EXPW,
                       preferred_element_type=jnp.float32).reshape(J, R, WF)
        rows.append(jnp.sum(a128 * xl_w, axis=0))
    return jnp.stack(rows, axis=0) + bias_w


def _body(x_ref, Wl1_ref, bl1_ref, Wr1_ref, br1_ref, A1_ref, bias1_ref,
          Wl2_ref, bl2_ref, Wr2_ref, br2_ref, A2_ref, bias2_ref,
          EXPW_ref, WoutR_ref, bout_ref, out_ref, *, Gb):
    R = Gb // KP
    EXPW = EXPW_ref[...]
    x = x_ref[...]
    h1w = _gat_wide(x.reshape(J * Gb, 3),
                    Wl1_ref[...], bl1_ref[...], Wr1_ref[...], br1_ref[...],
                    A1_ref[...], bias1_ref[...], EXPW, Gb)
    h1 = _narrow(_elu(h1w), R)
    h2w = _gat_wide(h1.reshape(J * Gb, F),
                    Wl2_ref[...], bl2_ref[...], Wr2_ref[...], br2_ref[...],
                    A2_ref[...], bias2_ref[...], EXPW, Gb)
    h2 = _narrow(_elu(h2w), R)
    acc = jnp.dot(h2[0], WoutR_ref[0], preferred_element_type=jnp.float32)
    for j in range(1, J):
        acc = acc + jnp.dot(h2[j], WoutR_ref[j],
                            preferred_element_type=jnp.float32)
    out_ref[...] = acc + bout_ref[...]


def kernel(x_seq, Wl1, bl1, Wr1, br1, att1, bias1,
           Wl2, bl2, Wr2, br2, att2, bias2, Wout, bout):
    B, T, _ = x_seq.shape
    G = B * T
    Gb = 256
    assert G % Gb == 0

    x3 = x_seq.reshape(G, J, 3).transpose(1, 0, 2)

    EXP = jnp.asarray(_EXP)
    EXPW = jnp.asarray(_EXP_W)
    A1 = att1.reshape(F, 1) * EXP.T * _LOG2E
    A2 = att2.reshape(F, 1) * EXP.T * _LOG2E
    A1w = jnp.kron(jnp.eye(KP, dtype=jnp.float32), A1)
    A2w = jnp.kron(jnp.eye(KP, dtype=jnp.float32), A2)
    bias1w = jnp.tile(bias1, (KP,)).reshape(1, 1, WF)
    bias2w = jnp.tile(bias2, (KP,)).reshape(1, 1, WF)
    WoutR = Wout.reshape(J, F, 256)

    full = lambda shape: pl.BlockSpec(shape, lambda g: (0,) * len(shape))
    out = pl.pallas_call(
        functools.partial(_body, Gb=Gb),
        grid=(G // Gb,),
        in_specs=[
            pl.BlockSpec((J, Gb, 3), lambda g: (0, g, 0)),
            full((3, F)), full((1, F)), full((3, F)), full((1, F)),
            full((WF, KH)), full((1, 1, WF)),
            full((F, F)), full((1, F)), full((F, F)), full((1, F)),
            full((WF, KH)), full((1, 1, WF)),
            full((KH, WF)), full((J, F, 256)), full((1, 256)),
        ],
        out_specs=pl.BlockSpec((Gb, 256), lambda g: (g, 0)),
        out_shape=jax.ShapeDtypeStruct((G, 256), jnp.float32),
    )(x3, Wl1, bl1.reshape(1, F), Wr1, br1.reshape(1, F), A1w, bias1w,
      Wl2, bl2.reshape(1, F), Wr2, br2.reshape(1, F), A2w, bias2w,
      EXPW, WoutR, bout.reshape(1, 256))
    return out.reshape(B, T, 256)
